# baseline (speedup 1.0000x reference)
import functools

import jax
import jax.numpy as jnp
from jax import lax
from jax.experimental import pallas as pl
from jax.experimental.pallas import tpu as pltpu


def _fold_bn(gamma, beta, mean, var, eps=1e-5):
    scale = gamma / jnp.sqrt(var + eps)
    return scale, beta - mean * scale


def _block_kernel(x_ref, w1_ref, w2_ref, bn_ref, bnc_ref, eye_ref, eyef_ref,
                  o_ref, cols_ref, *, H, W, C, B):
    HW = H * W
    M = B * HW
    x = x_ref[...].reshape(M, C)

    s1 = bn_ref[0:1, :]
    b1 = bn_ref[1:2, :]
    s2c = bnc_ref[:, 2:3]
    b2c = bnc_ref[:, 3:4]

    def fill_cols(yb):
        g = jnp.pad(yb.reshape(B, H, W, C), ((0, 0), (1, 1), (1, 1), (0, 0)))
        for t, (dy, dx) in enumerate((dy, dx) for dy in range(3)
                                     for dx in range(3)):
            cols_ref[:, t * C:(t + 1) * C] = (
                g[:, dy:dy + H, dx:dx + W, :].reshape(M, C))

    fill_cols(jnp.maximum(x * s1 + b1, 0.0).astype(jnp.bfloat16))
    acc1 = lax.dot_general(w1_ref[...], cols_ref[...],
                           (((0,), (1,)), ((), ())),
                           preferred_element_type=jnp.float32)

    y2 = jnp.maximum(acc1 * s2c + b2c, 0.0).astype(jnp.bfloat16)
    y2t = lax.dot_general(y2, eye_ref[...], (((0,), (0,)), ((), ())),
                          preferred_element_type=jnp.float32)
    fill_cols(y2t.astype(jnp.bfloat16))
    acc2 = lax.dot_general(w2_ref[...], cols_ref[...],
                           (((0,), (1,)), ((), ())),
                           preferred_element_type=jnp.float32)

    out = lax.dot_general(acc2, eyef_ref[...], (((0,), (0,)), ((), ())),
                          preferred_element_type=jnp.float32)
    o_ref[...] = (x + out).reshape(B, HW, C)


@jax.jit
def _basic_unit(x_nchw, w1, w2, bn1, bn2):
    n, c, h, w = x_nchw.shape
    hw = h * w
    x2d = jnp.transpose(x_nchw, (0, 2, 3, 1)).reshape(n, hw, c)

    s1, b1 = _fold_bn(*bn1)
    s2, b2 = _fold_bn(*bn2)
    bn = jnp.zeros((8, c), jnp.float32)
    bn = bn.at[0].set(s1).at[1].set(b1).at[2].set(s2).at[3].set(b2)

    def prep_w(wt):
        wk = jnp.transpose(wt, (2, 3, 1, 0)).reshape(9 * c, c)
        return wk.astype(jnp.bfloat16)

    w1k = prep_w(w1)
    w2k = prep_w(w2)
    bnc = jnp.transpose(bn)
    eye = jnp.eye(c, dtype=jnp.bfloat16)
    eyef = jnp.eye(c, dtype=jnp.float32)

    b = next(bb for bb in (8, 4, 2, 1) if n % bb == 0)
    kfn = functools.partial(_block_kernel, H=h, W=w, C=c, B=b)
    out2d = pl.pallas_call(
        kfn,
        out_shape=jax.ShapeDtypeStruct((n, hw, c), jnp.float32),
        grid=(n // b,),
        in_specs=[
            pl.BlockSpec((b, hw, c), lambda i: (i, 0, 0)),
            pl.BlockSpec((9 * c, c), lambda i: (0, 0)),
            pl.BlockSpec((9 * c, c), lambda i: (0, 0)),
            pl.BlockSpec((8, c), lambda i: (0, 0)),
            pl.BlockSpec((c, 8), lambda i: (0, 0)),
            pl.BlockSpec((c, c), lambda i: (0, 0)),
            pl.BlockSpec((c, c), lambda i: (0, 0)),
        ],
        out_specs=pl.BlockSpec((b, hw, c), lambda i: (i, 0, 0)),
        scratch_shapes=[pltpu.VMEM((b * hw, 9 * c), jnp.bfloat16)],
        compiler_params=pltpu.CompilerParams(
            dimension_semantics=("parallel",),
            vmem_limit_bytes=64 * 1024 * 1024,
        ),
    )(x2d, w1k, w2k, bn, bnc, eye, eyef)

    out = out2d.reshape(n, h, w, c)
    return jnp.transpose(out, (0, 3, 1, 2))


def kernel(x, w1, w2, bn1_gamma, bn1_beta, bn1_mean, bn1_var,
           bn2_gamma, bn2_beta, bn2_mean, bn2_var):
    return _basic_unit(x, w1, w2,
                       (bn1_gamma, bn1_beta, bn1_mean, bn1_var),
                       (bn2_gamma, bn2_beta, bn2_mean, bn2_var))


# speedup vs baseline: 1.0052x; 1.0052x over previous
import functools

import jax
import jax.numpy as jnp
from jax import lax
from jax.experimental import pallas as pl
from jax.experimental.pallas import tpu as pltpu


def _fold_bn(gamma, beta, mean, var, eps=1e-5):
    scale = gamma / jnp.sqrt(var + eps)
    return scale, beta - mean * scale


def _block_kernel(x_ref, w1_ref, w2_ref, bn_ref, bnc_ref, eye_ref, eyef_ref,
                  o_ref, cols_ref, *, H, W, C, B):
    HW = H * W
    M = B * HW
    x = x_ref[...].reshape(M, C)

    s1 = bn_ref[0:1, :]
    b1 = bn_ref[1:2, :]
    s2c = bnc_ref[:, 2:3]
    b2c = bnc_ref[:, 3:4]

    def fill_cols(yb):
        g = jnp.pad(yb.reshape(B, H, W, C), ((0, 0), (1, 1), (1, 1), (0, 0)))
        for t, (dy, dx) in enumerate((dy, dx) for dy in range(3)
                                     for dx in range(3)):
            cols_ref[:, t * C:(t + 1) * C] = (
                g[:, dy:dy + H, dx:dx + W, :].reshape(M, C))

    fill_cols(jnp.maximum(x * s1 + b1, 0.0).astype(jnp.bfloat16))
    acc1 = lax.dot_general(w1_ref[...], cols_ref[...],
                           (((0,), (1,)), ((), ())),
                           preferred_element_type=jnp.float32)

    y2 = jnp.maximum(acc1 * s2c + b2c, 0.0).astype(jnp.bfloat16)
    y2t = lax.dot_general(y2, eye_ref[...], (((0,), (0,)), ((), ())),
                          preferred_element_type=jnp.float32)
    fill_cols(y2t.astype(jnp.bfloat16))
    acc2 = lax.dot_general(w2_ref[...], cols_ref[...],
                           (((0,), (1,)), ((), ())),
                           preferred_element_type=jnp.float32)

    out = lax.dot_general(acc2, eyef_ref[...], (((0,), (0,)), ((), ())),
                          preferred_element_type=jnp.float32)
    o_ref[...] = (x + out).reshape(B, HW, C)


@jax.jit
def _basic_unit(x_nchw, w1, w2, bn1, bn2):
    n, c, h, w = x_nchw.shape
    hw = h * w
    x2d = jnp.transpose(x_nchw, (0, 2, 3, 1)).reshape(n, hw, c)

    s1, b1 = _fold_bn(*bn1)
    s2, b2 = _fold_bn(*bn2)
    bn = jnp.zeros((8, c), jnp.float32)
    bn = bn.at[0].set(s1).at[1].set(b1).at[2].set(s2).at[3].set(b2)

    def prep_w(wt):
        wk = jnp.transpose(wt, (2, 3, 1, 0)).reshape(9 * c, c)
        return wk.astype(jnp.bfloat16)

    w1k = prep_w(w1)
    w2k = prep_w(w2)
    bnc = jnp.transpose(bn)
    eye = jnp.eye(c, dtype=jnp.bfloat16)
    eyef = jnp.eye(c, dtype=jnp.float32)

    b = next(bb for bb in (4, 2, 1) if n % bb == 0)
    kfn = functools.partial(_block_kernel, H=h, W=w, C=c, B=b)
    out2d = pl.pallas_call(
        kfn,
        out_shape=jax.ShapeDtypeStruct((n, hw, c), jnp.float32),
        grid=(n // b,),
        in_specs=[
            pl.BlockSpec((b, hw, c), lambda i: (i, 0, 0)),
            pl.BlockSpec((9 * c, c), lambda i: (0, 0)),
            pl.BlockSpec((9 * c, c), lambda i: (0, 0)),
            pl.BlockSpec((8, c), lambda i: (0, 0)),
            pl.BlockSpec((c, 8), lambda i: (0, 0)),
            pl.BlockSpec((c, c), lambda i: (0, 0)),
            pl.BlockSpec((c, c), lambda i: (0, 0)),
        ],
        out_specs=pl.BlockSpec((b, hw, c), lambda i: (i, 0, 0)),
        scratch_shapes=[pltpu.VMEM((b * hw, 9 * c), jnp.bfloat16)],
        compiler_params=pltpu.CompilerParams(
            dimension_semantics=("parallel",),
            vmem_limit_bytes=64 * 1024 * 1024,
        ),
    )(x2d, w1k, w2k, bn, bnc, eye, eyef)

    out = out2d.reshape(n, h, w, c)
    return jnp.transpose(out, (0, 3, 1, 2))


def kernel(x, w1, w2, bn1_gamma, bn1_beta, bn1_mean, bn1_var,
           bn2_gamma, bn2_beta, bn2_mean, bn2_var):
    return _basic_unit(x, w1, w2,
                       (bn1_gamma, bn1_beta, bn1_mean, bn1_var),
                       (bn2_gamma, bn2_beta, bn2_mean, bn2_var))
